# Initial kernel scaffold; baseline (speedup 1.0000x reference)
import jax
import jax.numpy as jnp
from jax.experimental import pallas as pl


def kernel(x, edge_index, logp, means, logvars, W1, b1, W2, b2, W3, b3):
    raise NotImplementedError("write your pallas kernel here")


# speedup vs baseline: 100.8290x; 100.8290x over previous
import functools

import jax
import jax.numpy as jnp
import numpy as np
from jax import lax
from jax.experimental import pallas as pl
from jax.experimental.pallas import tpu as pltpu
from jax.experimental.pallas import tpu_sc as plsc

_F32 = jnp.float32
_NTILES = 16
_NCORES = 2
_LANE = 128


def _make_deg_kernel(n_pad, gt):
    npt = n_pad // _NTILES
    mesh = plsc.VectorSubcoreMesh(core_axis_name="c", subcore_axis_name="s")

    @functools.partial(
        pl.kernel,
        out_type=jax.ShapeDtypeStruct((_NCORES * n_pad,), _F32),
        mesh=mesh,
        scratch_types=[
            pltpu.VMEM((gt, _LANE), jnp.int32),
            pltpu.VMEM((_LANE,), _F32),
            pltpu.VMEM((npt,), _F32),
            pltpu.VMEM_SHARED((n_pad,), _F32),
        ],
    )
    def deg_kernel(dst_hbm, ones_hbm, out_hbm, idx_v, ones_v, stage_v, acc_sh):
        cid = lax.axis_index("c")
        sid = lax.axis_index("s")

        def zbody(i, carry):
            stage_v[pl.ds(i * 16, 16)] = jnp.zeros((16,), _F32)
            return carry

        lax.fori_loop(0, npt // 16, zbody, 0)
        pltpu.sync_copy(stage_v, acc_sh.at[pl.ds(sid * npt, npt)])
        pltpu.sync_copy(ones_hbm, ones_v)
        pltpu.sync_copy(dst_hbm.at[pl.ds((cid * _NTILES + sid) * gt, gt)], idx_v)
        plsc.subcore_barrier()

        def body(g, carry):
            pltpu.sync_copy(ones_v, acc_sh.at[idx_v.at[g]], add=True)
            return carry

        lax.fori_loop(0, gt, body, 0)
        plsc.subcore_barrier()
        pltpu.sync_copy(acc_sh.at[pl.ds(sid * npt, npt)], stage_v)
        pltpu.sync_copy(stage_v, out_hbm.at[pl.ds(cid * n_pad + sid * npt, npt)])

    return deg_kernel


_CG = 40


def _make_prop_kernel(n_pad, gt, width):
    npt = n_pad // _NTILES
    mesh = plsc.VectorSubcoreMesh(core_axis_name="c", subcore_axis_name="s")

    @functools.partial(
        pl.kernel,
        out_type=jax.ShapeDtypeStruct((_NCORES, n_pad, width), _F32),
        mesh=mesh,
        scratch_types=[
            pltpu.VMEM((_CG, _LANE), jnp.int32),
            pltpu.VMEM((_CG, _LANE), jnp.int32),
            pltpu.VMEM((_LANE, width), _F32),
            pltpu.VMEM_SHARED((n_pad, width), _F32),
            pltpu.SemaphoreType.DMA,
        ],
        compiler_params=pltpu.CompilerParams(use_tc_tiling_on_sc=False),
    )
    def prop_kernel(h0_hbm, h1_hbm, src_hbm, dst_hbm, out_hbm,
                    src_v, dst_v, rows_v, acc_sh, sem):
        cid = lax.axis_index("c")
        sid = lax.axis_index("s")

        def zbody(i, carry):
            for j in range(width // 16):
                rows_v[i, pl.ds(j * 16, 16)] = jnp.zeros((16,), _F32)
            return carry

        lax.fori_loop(0, _LANE, zbody, 0)
        for c in range(npt // _LANE):
            pltpu.sync_copy(rows_v,
                            acc_sh.at[pl.ds(sid * npt + c * _LANE, _LANE)])
        plsc.subcore_barrier()

        def edge_sweep(h_hbm):
            for c in range(gt // _CG):
                base = sid * gt + c * _CG
                pltpu.sync_copy(src_hbm.at[pl.ds(base, _CG)], src_v)
                pltpu.sync_copy(dst_hbm.at[pl.ds(base, _CG)], dst_v)

                def body(g, carry):
                    pltpu.async_copy(h_hbm.at[src_v.at[g]], rows_v, sem).wait()
                    pltpu.sync_copy(rows_v, acc_sh.at[dst_v.at[g]], add=True)
                    return carry

                lax.fori_loop(0, _CG, body, 0)

        @pl.when(cid == 0)
        def _():
            edge_sweep(h0_hbm)

        @pl.when(cid == 1)
        def _():
            edge_sweep(h1_hbm)

        plsc.subcore_barrier()
        for c in range(npt // _LANE):
            rs = pl.ds(sid * npt + c * _LANE, _LANE)
            pltpu.sync_copy(acc_sh.at[rs], rows_v)
            pltpu.sync_copy(rows_v, out_hbm.at[cid, rs])

    return prop_kernel


def _kernel_a(x_ref, degp_ref, w1_ref, b1_ref, h1a_ref, h1b_ref, dinv_ref,
              *, blk, n):
    i = pl.program_id(0)
    xb = x_ref[...]
    isn = jnp.isnan(xb)
    s = isn.astype(_F32)
    xc = jnp.where(isn, 0.0, xb)
    b0 = jnp.dot(xc, w1_ref[...], preferred_element_type=_F32) + b1_ref[...]
    deg = degp_ref[0] + degp_ref[1] + 1.0
    dinv = lax.rsqrt(jnp.maximum(deg, 1.0))
    rows = i * blk + lax.broadcasted_iota(jnp.int32, (blk, 1), 0)
    mask = rows < n
    ha = jnp.concatenate([b0, s[:, :32]], axis=1) * dinv
    hb = s[:, 32:] * dinv
    h1a_ref[...] = jnp.where(mask, ha, 0.0)
    h1b_ref[...] = jnp.where(mask, hb, 0.0)
    dinv_ref[...] = dinv


def _kernel_b(x_ref, g_ref, h1a_ref, h1b_ref, dinv_ref, w1_ref, w2_ref,
              means_ref, logvars_ref, logp_ref, h2a_ref, h2b_ref,
              *, blk, n, k):
    i = pl.program_id(0)
    p0 = (g_ref[0] + h1a_ref[...]) * dinv_ref[...]
    p1 = (g_ref[1] + h1b_ref[...]) * dinv_ref[...]
    px = p0[:, :64]
    ps = jnp.concatenate([p0[:, 64:], p1], axis=1)
    xb = x_ref[...]
    isn = jnp.isnan(xb)
    s = isn.astype(_F32)
    xc = jnp.where(isn, 0.0, xb)
    notn = 1.0 - s
    means = means_ref[...]
    logvars = logvars_ref[...]
    variances = jnp.exp(logvars)
    inv_var = 1.0 / variances
    u = notn * xc * xc
    v = notn * xc
    log_n = -0.5 * (
        jnp.dot(u, inv_var.T, preferred_element_type=_F32)
        - 2.0 * jnp.dot(v, (means * inv_var).T, preferred_element_type=_F32)
        + jnp.dot(notn, (means * means * inv_var).T, preferred_element_type=_F32)
    ) - 0.5 * jnp.sum(logvars, axis=-1)[None, :]
    z = logp_ref[...] + log_n
    z = z - jnp.max(z, axis=1, keepdims=True)
    ez = jnp.exp(z)
    gamma = ez / jnp.sum(ez, axis=1, keepdims=True)

    w1 = w1_ref[...]
    w1sq = w1 * w1
    inv_sqrt2 = np.float32(1.0 / np.sqrt(2.0))
    inv_sqrt2pi = np.float32(1.0 / np.sqrt(2.0 * np.pi))
    x1 = jnp.zeros_like(px)
    for kk in range(k):
        conv_x = px + jnp.dot(ps * means[kk][None, :], w1,
                              preferred_element_type=_F32)
        conv_c = jnp.dot(ps * variances[kk][None, :], w1sq,
                         preferred_element_type=_F32)
        sigma = jnp.sqrt(conv_c + 1e-8)
        ratio = conv_x / sigma
        cdf = 0.5 * (1.0 + lax.erf(ratio * inv_sqrt2))
        pdf = jnp.exp(-0.5 * ratio * ratio) * inv_sqrt2pi
        e_relu = conv_x * cdf + sigma * pdf
        x1 = x1 + gamma[:, kk][:, None] * e_relu

    h2s = jnp.dot(x1, w2_ref[...], preferred_element_type=_F32) * dinv_ref[...]
    rows = i * blk + lax.broadcasted_iota(jnp.int32, (blk, 1), 0)
    h2s = jnp.where(rows < n, h2s, 0.0)
    h2a_ref[...] = h2s[:, :32]
    h2b_ref[...] = h2s[:, 32:]


def _kernel_c(g_ref, h2a_ref, h2b_ref, dinv_ref, b2_ref, w3_ref,
              x2_ref, h3a_ref, h3b_ref, *, blk, n):
    i = pl.program_id(0)
    pre = jnp.concatenate([g_ref[0] + h2a_ref[...],
                           g_ref[1] + h2b_ref[...]], axis=1)
    x2 = pre * dinv_ref[...] + b2_ref[...]
    x2 = jnp.maximum(x2, 0.0)
    x2_ref[...] = x2
    h3s = jnp.dot(x2, w3_ref[...], preferred_element_type=_F32) * dinv_ref[...]
    rows = i * blk + lax.broadcasted_iota(jnp.int32, (blk, 1), 0)
    h3s = jnp.where(rows < n, h3s, 0.0)
    h3a_ref[...] = h3s[:, :32]
    h3b_ref[...] = h3s[:, 32:]


def _kernel_d(g_ref, h3a_ref, h3b_ref, dinv_ref, b3_ref, out_ref, *, ncls):
    pre = jnp.concatenate([g_ref[0] + h3a_ref[...],
                           g_ref[1] + h3b_ref[...]], axis=1)
    x3 = pre * dinv_ref[...] + b3_ref[...]
    cols = lax.broadcasted_iota(jnp.int32, x3.shape, 1)
    x3 = jnp.where(cols < ncls, x3, -1e30)
    m = jnp.max(x3, axis=1, keepdims=True)
    ex = jnp.exp(x3 - m)
    lse = jnp.log(jnp.sum(ex, axis=1, keepdims=True))
    out_ref[...] = x3 - m - lse


def kernel(x, edge_index, logp, means, logvars, W1, b1, W2, b2, W3, b3):
    n, f_in = x.shape
    hid = W1.shape[1]
    ncls = W3.shape[1]
    k = means.shape[0]
    e = edge_index.shape[1]

    n_pad = ((n + 16 + 2047) // 2048) * 2048
    gt = -(-e // (_NTILES * _LANE))
    gt = ((gt + 2 * _CG - 1) // (2 * _CG)) * (2 * _CG)
    e_pad = _NTILES * gt * _LANE
    gtd = gt // 2
    half = (hid + f_in) // 2

    fill = (jnp.arange(e_pad - e, dtype=jnp.int32) % 16) + n
    src2d = jnp.concatenate([edge_index[0], fill]).reshape(-1, _LANE)
    dst2d = jnp.concatenate([edge_index[1], fill]).reshape(-1, _LANE)
    x_pad = jnp.pad(x, ((0, n_pad - n), (0, 0)))
    ones128 = jnp.ones((_LANE,), _F32)
    b1r = b1.reshape(1, hid)
    b2r = b2.reshape(1, hid)
    b3r = jnp.pad(b3, (0, hid - ncls)).reshape(1, hid)
    w3p = jnp.pad(W3, ((0, 0), (0, hid - ncls)))
    logp_r = logp.reshape(1, k)

    degp = _make_deg_kernel(n_pad, gtd)(dst2d, ones128)
    degp3 = degp.reshape(_NCORES, n_pad)[:, :, None]

    nb = 8
    blk = n_pad // nb
    vspec = lambda w: pl.BlockSpec((blk, w), lambda i: (i, 0))
    cspec = pl.BlockSpec((_NCORES, blk, 1), lambda i: (0, i, 0))
    gspec = lambda w: pl.BlockSpec((_NCORES, blk, w), lambda i: (0, i, 0))
    full = lambda *s: pl.BlockSpec(s, lambda i: tuple(0 for _ in s))

    h1a, h1b, dinv = pl.pallas_call(
        functools.partial(_kernel_a, blk=blk, n=n),
        grid=(nb,),
        in_specs=[vspec(f_in), cspec, full(f_in, hid), full(1, hid)],
        out_specs=[vspec(half), vspec(half), vspec(1)],
        out_shape=[jax.ShapeDtypeStruct((n_pad, half), _F32),
                   jax.ShapeDtypeStruct((n_pad, half), _F32),
                   jax.ShapeDtypeStruct((n_pad, 1), _F32)],
    )(x_pad, degp3, W1, b1r)

    g1 = _make_prop_kernel(n_pad, gt, half)(h1a, h1b, src2d, dst2d)

    h2a, h2b = pl.pallas_call(
        functools.partial(_kernel_b, blk=blk, n=n, k=k),
        grid=(nb,),
        in_specs=[vspec(f_in), gspec(half), vspec(half), vspec(half),
                  vspec(1), full(f_in, hid), full(hid, hid), full(k, f_in),
                  full(k, f_in), full(1, k)],
        out_specs=[vspec(hid // 2), vspec(hid // 2)],
        out_shape=[jax.ShapeDtypeStruct((n_pad, hid // 2), _F32),
                   jax.ShapeDtypeStruct((n_pad, hid // 2), _F32)],
    )(x_pad, g1, h1a, h1b, dinv, W1, W2, means, logvars, logp_r)

    g2 = _make_prop_kernel(n_pad, gt, hid // 2)(h2a, h2b, src2d, dst2d)

    x2p, h3a, h3b = pl.pallas_call(
        functools.partial(_kernel_c, blk=blk, n=n),
        grid=(nb,),
        in_specs=[gspec(hid // 2), vspec(hid // 2), vspec(hid // 2),
                  vspec(1), full(1, hid), full(hid, hid)],
        out_specs=[vspec(hid), vspec(hid // 2), vspec(hid // 2)],
        out_shape=[jax.ShapeDtypeStruct((n_pad, hid), _F32),
                   jax.ShapeDtypeStruct((n_pad, hid // 2), _F32),
                   jax.ShapeDtypeStruct((n_pad, hid // 2), _F32)],
    )(g2, h2a, h2b, dinv, b2r, w3p)

    g3 = _make_prop_kernel(n_pad, gt, hid // 2)(h3a, h3b, src2d, dst2d)

    logits = pl.pallas_call(
        functools.partial(_kernel_d, ncls=ncls),
        grid=(nb,),
        in_specs=[gspec(hid // 2), vspec(hid // 2), vspec(hid // 2),
                  vspec(1), full(1, hid)],
        out_specs=vspec(hid),
        out_shape=jax.ShapeDtypeStruct((n_pad, hid), _F32),
    )(g3, h3a, h3b, dinv, b3r)

    return (logits[:n, :ncls], x2p[:n])


# speedup vs baseline: 119.6665x; 1.1868x over previous
import functools

import jax
import jax.numpy as jnp
import numpy as np
from jax import lax
from jax.experimental import pallas as pl
from jax.experimental.pallas import tpu as pltpu
from jax.experimental.pallas import tpu_sc as plsc

_F32 = jnp.float32
_NTILES = 16
_NCORES = 2
_LANE = 128


def _make_deg_kernel(n_pad, gt):
    npt = n_pad // _NTILES
    mesh = plsc.VectorSubcoreMesh(core_axis_name="c", subcore_axis_name="s")

    @functools.partial(
        pl.kernel,
        out_type=jax.ShapeDtypeStruct((_NCORES * n_pad,), _F32),
        mesh=mesh,
        scratch_types=[
            pltpu.VMEM((gt, _LANE), jnp.int32),
            pltpu.VMEM((_LANE,), _F32),
            pltpu.VMEM((npt,), _F32),
            pltpu.VMEM_SHARED((n_pad,), _F32),
        ],
    )
    def deg_kernel(dst_hbm, ones_hbm, out_hbm, idx_v, ones_v, stage_v, acc_sh):
        cid = lax.axis_index("c")
        sid = lax.axis_index("s")

        def zbody(i, carry):
            stage_v[pl.ds(i * 16, 16)] = jnp.zeros((16,), _F32)
            return carry

        lax.fori_loop(0, npt // 16, zbody, 0)
        pltpu.sync_copy(stage_v, acc_sh.at[pl.ds(sid * npt, npt)])
        pltpu.sync_copy(ones_hbm, ones_v)
        pltpu.sync_copy(dst_hbm.at[pl.ds((cid * _NTILES + sid) * gt, gt)], idx_v)
        plsc.subcore_barrier()

        def body(g, carry):
            pltpu.sync_copy(ones_v, acc_sh.at[idx_v.at[g]], add=True)
            return carry

        lax.fori_loop(0, gt, body, 0)
        plsc.subcore_barrier()
        pltpu.sync_copy(acc_sh.at[pl.ds(sid * npt, npt)], stage_v)
        pltpu.sync_copy(stage_v, out_hbm.at[pl.ds(cid * n_pad + sid * npt, npt)])

    return deg_kernel


_CG = 40


def _make_prop_kernel(n_pad, gt, width):
    npt = n_pad // _NTILES
    mesh = plsc.VectorSubcoreMesh(core_axis_name="c", subcore_axis_name="s")

    @functools.partial(
        pl.kernel,
        out_type=jax.ShapeDtypeStruct((_NCORES, n_pad, width), _F32),
        mesh=mesh,
        scratch_types=[
            pltpu.VMEM((_CG, _LANE), jnp.int32),
            pltpu.VMEM((_CG, _LANE), jnp.int32),
            pltpu.VMEM((2 * _LANE, width), _F32),
            pltpu.VMEM_SHARED((n_pad, width), _F32),
            pltpu.SemaphoreType.DMA,
            pltpu.SemaphoreType.DMA,
        ],
        compiler_params=pltpu.CompilerParams(use_tc_tiling_on_sc=False),
    )
    def prop_kernel(h0_hbm, h1_hbm, src_hbm, dst_hbm, out_hbm,
                    src_v, dst_v, rows_v, acc_sh, sem0, sem1):
        cid = lax.axis_index("c")
        sid = lax.axis_index("s")

        buf0 = rows_v.at[pl.ds(0, _LANE)]
        buf1 = rows_v.at[pl.ds(_LANE, _LANE)]

        def zbody(i, carry):
            for j in range(width // 16):
                rows_v[i, pl.ds(j * 16, 16)] = jnp.zeros((16,), _F32)
            return carry

        lax.fori_loop(0, _LANE, zbody, 0)
        for c in range(npt // _LANE):
            pltpu.sync_copy(buf0,
                            acc_sh.at[pl.ds(sid * npt + c * _LANE, _LANE)])
        plsc.subcore_barrier()

        def edge_sweep(h_hbm):
            for c in range(gt // _CG):
                base = sid * gt + c * _CG
                pltpu.sync_copy(src_hbm.at[pl.ds(base, _CG)], src_v)
                pltpu.sync_copy(dst_hbm.at[pl.ds(base, _CG)], dst_v)
                pltpu.async_copy(h_hbm.at[src_v.at[0]], buf0, sem0)

                def pair(i, carry):
                    g0 = 2 * i
                    pltpu.make_async_copy(h_hbm.at[src_v.at[g0]],
                                          buf0, sem0).wait()
                    pltpu.async_copy(h_hbm.at[src_v.at[g0 + 1]], buf1, sem1)
                    pltpu.sync_copy(buf0, acc_sh.at[dst_v.at[g0]], add=True)
                    pltpu.make_async_copy(h_hbm.at[src_v.at[g0 + 1]],
                                          buf1, sem1).wait()

                    @pl.when(g0 + 2 < _CG)
                    def _():
                        pltpu.async_copy(h_hbm.at[src_v.at[g0 + 2]],
                                         buf0, sem0)

                    pltpu.sync_copy(buf1, acc_sh.at[dst_v.at[g0 + 1]],
                                    add=True)
                    return carry

                lax.fori_loop(0, _CG // 2, pair, 0)

        @pl.when(cid == 0)
        def _():
            edge_sweep(h0_hbm)

        @pl.when(cid == 1)
        def _():
            edge_sweep(h1_hbm)

        plsc.subcore_barrier()
        for c in range(npt // _LANE):
            rs = pl.ds(sid * npt + c * _LANE, _LANE)
            pltpu.sync_copy(acc_sh.at[rs], buf0)
            pltpu.sync_copy(buf0, out_hbm.at[cid, rs])

    return prop_kernel


def _kernel_a(x_ref, degp_ref, w1_ref, b1_ref, h1a_ref, h1b_ref, dinv_ref,
              *, blk, n):
    i = pl.program_id(0)
    xb = x_ref[...]
    isn = jnp.isnan(xb)
    s = isn.astype(_F32)
    xc = jnp.where(isn, 0.0, xb)
    b0 = jnp.dot(xc, w1_ref[...], preferred_element_type=_F32) + b1_ref[...]
    deg = degp_ref[0] + degp_ref[1] + 1.0
    dinv = lax.rsqrt(jnp.maximum(deg, 1.0))
    rows = i * blk + lax.broadcasted_iota(jnp.int32, (blk, 1), 0)
    mask = rows < n
    ha = jnp.concatenate([b0, s[:, :32]], axis=1) * dinv
    hb = s[:, 32:] * dinv
    h1a_ref[...] = jnp.where(mask, ha, 0.0)
    h1b_ref[...] = jnp.where(mask, hb, 0.0)
    dinv_ref[...] = dinv


def _kernel_b(x_ref, g_ref, h1a_ref, h1b_ref, dinv_ref, w1_ref, w2_ref,
              means_ref, logvars_ref, logp_ref, h2a_ref, h2b_ref,
              *, blk, n, k):
    i = pl.program_id(0)
    p0 = (g_ref[0] + h1a_ref[...]) * dinv_ref[...]
    p1 = (g_ref[1] + h1b_ref[...]) * dinv_ref[...]
    px = p0[:, :64]
    ps = jnp.concatenate([p0[:, 64:], p1], axis=1)
    xb = x_ref[...]
    isn = jnp.isnan(xb)
    s = isn.astype(_F32)
    xc = jnp.where(isn, 0.0, xb)
    notn = 1.0 - s
    means = means_ref[...]
    logvars = logvars_ref[...]
    variances = jnp.exp(logvars)
    inv_var = 1.0 / variances
    u = notn * xc * xc
    v = notn * xc
    log_n = -0.5 * (
        jnp.dot(u, inv_var.T, preferred_element_type=_F32)
        - 2.0 * jnp.dot(v, (means * inv_var).T, preferred_element_type=_F32)
        + jnp.dot(notn, (means * means * inv_var).T, preferred_element_type=_F32)
    ) - 0.5 * jnp.sum(logvars, axis=-1)[None, :]
    z = logp_ref[...] + log_n
    z = z - jnp.max(z, axis=1, keepdims=True)
    ez = jnp.exp(z)
    gamma = ez / jnp.sum(ez, axis=1, keepdims=True)

    w1 = w1_ref[...]
    w1sq = w1 * w1
    inv_sqrt2 = np.float32(1.0 / np.sqrt(2.0))
    inv_sqrt2pi = np.float32(1.0 / np.sqrt(2.0 * np.pi))
    x1 = jnp.zeros_like(px)
    for kk in range(k):
        conv_x = px + jnp.dot(ps * means[kk][None, :], w1,
                              preferred_element_type=_F32)
        conv_c = jnp.dot(ps * variances[kk][None, :], w1sq,
                         preferred_element_type=_F32)
        sigma = jnp.sqrt(conv_c + 1e-8)
        ratio = conv_x / sigma
        cdf = 0.5 * (1.0 + lax.erf(ratio * inv_sqrt2))
        pdf = jnp.exp(-0.5 * ratio * ratio) * inv_sqrt2pi
        e_relu = conv_x * cdf + sigma * pdf
        x1 = x1 + gamma[:, kk][:, None] * e_relu

    h2s = jnp.dot(x1, w2_ref[...], preferred_element_type=_F32) * dinv_ref[...]
    rows = i * blk + lax.broadcasted_iota(jnp.int32, (blk, 1), 0)
    h2s = jnp.where(rows < n, h2s, 0.0)
    h2a_ref[...] = h2s[:, :32]
    h2b_ref[...] = h2s[:, 32:]


def _kernel_c(g_ref, h2a_ref, h2b_ref, dinv_ref, b2_ref, w3_ref,
              x2_ref, h3a_ref, h3b_ref, *, blk, n):
    i = pl.program_id(0)
    pre = jnp.concatenate([g_ref[0] + h2a_ref[...],
                           g_ref[1] + h2b_ref[...]], axis=1)
    x2 = pre * dinv_ref[...] + b2_ref[...]
    x2 = jnp.maximum(x2, 0.0)
    x2_ref[...] = x2
    h3s = jnp.dot(x2, w3_ref[...], preferred_element_type=_F32) * dinv_ref[...]
    rows = i * blk + lax.broadcasted_iota(jnp.int32, (blk, 1), 0)
    h3s = jnp.where(rows < n, h3s, 0.0)
    h3a_ref[...] = h3s[:, :32]
    h3b_ref[...] = h3s[:, 32:]


def _kernel_d(g_ref, h3a_ref, h3b_ref, dinv_ref, b3_ref, out_ref, *, ncls):
    pre = jnp.concatenate([g_ref[0] + h3a_ref[...],
                           g_ref[1] + h3b_ref[...]], axis=1)
    x3 = pre * dinv_ref[...] + b3_ref[...]
    cols = lax.broadcasted_iota(jnp.int32, x3.shape, 1)
    x3 = jnp.where(cols < ncls, x3, -1e30)
    m = jnp.max(x3, axis=1, keepdims=True)
    ex = jnp.exp(x3 - m)
    lse = jnp.log(jnp.sum(ex, axis=1, keepdims=True))
    out_ref[...] = x3 - m - lse


def kernel(x, edge_index, logp, means, logvars, W1, b1, W2, b2, W3, b3):
    n, f_in = x.shape
    hid = W1.shape[1]
    ncls = W3.shape[1]
    k = means.shape[0]
    e = edge_index.shape[1]

    n_pad = ((n + 16 + 2047) // 2048) * 2048
    gt = -(-e // (_NTILES * _LANE))
    gt = ((gt + 2 * _CG - 1) // (2 * _CG)) * (2 * _CG)
    e_pad = _NTILES * gt * _LANE
    gtd = gt // 2
    half = (hid + f_in) // 2

    fill = (jnp.arange(e_pad - e, dtype=jnp.int32) % 16) + n
    src2d = jnp.concatenate([edge_index[0], fill]).reshape(-1, _LANE)
    dst2d = jnp.concatenate([edge_index[1], fill]).reshape(-1, _LANE)
    x_pad = jnp.pad(x, ((0, n_pad - n), (0, 0)))
    ones128 = jnp.ones((_LANE,), _F32)
    b1r = b1.reshape(1, hid)
    b2r = b2.reshape(1, hid)
    b3r = jnp.pad(b3, (0, hid - ncls)).reshape(1, hid)
    w3p = jnp.pad(W3, ((0, 0), (0, hid - ncls)))
    logp_r = logp.reshape(1, k)

    degp = _make_deg_kernel(n_pad, gtd)(dst2d, ones128)
    degp3 = degp.reshape(_NCORES, n_pad)[:, :, None]

    nb = 8
    blk = n_pad // nb
    vspec = lambda w: pl.BlockSpec((blk, w), lambda i: (i, 0))
    cspec = pl.BlockSpec((_NCORES, blk, 1), lambda i: (0, i, 0))
    gspec = lambda w: pl.BlockSpec((_NCORES, blk, w), lambda i: (0, i, 0))
    full = lambda *s: pl.BlockSpec(s, lambda i: tuple(0 for _ in s))

    h1a, h1b, dinv = pl.pallas_call(
        functools.partial(_kernel_a, blk=blk, n=n),
        grid=(nb,),
        in_specs=[vspec(f_in), cspec, full(f_in, hid), full(1, hid)],
        out_specs=[vspec(half), vspec(half), vspec(1)],
        out_shape=[jax.ShapeDtypeStruct((n_pad, half), _F32),
                   jax.ShapeDtypeStruct((n_pad, half), _F32),
                   jax.ShapeDtypeStruct((n_pad, 1), _F32)],
    )(x_pad, degp3, W1, b1r)

    g1 = _make_prop_kernel(n_pad, gt, half)(h1a, h1b, src2d, dst2d)

    h2a, h2b = pl.pallas_call(
        functools.partial(_kernel_b, blk=blk, n=n, k=k),
        grid=(nb,),
        in_specs=[vspec(f_in), gspec(half), vspec(half), vspec(half),
                  vspec(1), full(f_in, hid), full(hid, hid), full(k, f_in),
                  full(k, f_in), full(1, k)],
        out_specs=[vspec(hid // 2), vspec(hid // 2)],
        out_shape=[jax.ShapeDtypeStruct((n_pad, hid // 2), _F32),
                   jax.ShapeDtypeStruct((n_pad, hid // 2), _F32)],
    )(x_pad, g1, h1a, h1b, dinv, W1, W2, means, logvars, logp_r)

    g2 = _make_prop_kernel(n_pad, gt, hid // 2)(h2a, h2b, src2d, dst2d)

    x2p, h3a, h3b = pl.pallas_call(
        functools.partial(_kernel_c, blk=blk, n=n),
        grid=(nb,),
        in_specs=[gspec(hid // 2), vspec(hid // 2), vspec(hid // 2),
                  vspec(1), full(1, hid), full(hid, hid)],
        out_specs=[vspec(hid), vspec(hid // 2), vspec(hid // 2)],
        out_shape=[jax.ShapeDtypeStruct((n_pad, hid), _F32),
                   jax.ShapeDtypeStruct((n_pad, hid // 2), _F32),
                   jax.ShapeDtypeStruct((n_pad, hid // 2), _F32)],
    )(g2, h2a, h2b, dinv, b2r, w3p)

    g3 = _make_prop_kernel(n_pad, gt, hid // 2)(h3a, h3b, src2d, dst2d)

    logits = pl.pallas_call(
        functools.partial(_kernel_d, ncls=ncls),
        grid=(nb,),
        in_specs=[gspec(hid // 2), vspec(hid // 2), vspec(hid // 2),
                  vspec(1), full(1, hid)],
        out_specs=vspec(hid),
        out_shape=jax.ShapeDtypeStruct((n_pad, hid), _F32),
    )(g3, h3a, h3b, dinv, b3r)

    return (logits[:n, :ncls], x2p[:n])
